# Initial kernel scaffold; baseline (speedup 1.0000x reference)
import jax
import jax.numpy as jnp
from jax.experimental import pallas as pl


def kernel(x, y, in_map, up__w, up__b, up__gamma, up__beta, up__mean, up__var, up2__w, up2__b, up2__gamma, up2__beta, up2__mean, up2__var, output_map__w, output_map__b, fp__cr1__w, fp__cr1__b, fp__cr1__gamma, fp__cr1__beta, fp__cr1__mean, fp__cr1__var, fp__cr2__w, fp__cr2__b, fp__cr2__gamma, fp__cr2__beta, fp__cr2__mean, fp__cr2__var, fp__cr3__w, fp__cr3__b, fp__cr3__gamma, fp__cr3__beta, fp__cr3__mean, fp__cr3__var, fp__cr4__w, fp__cr4__b, fp__cr4__gamma, fp__cr4__beta, fp__cr4__mean, fp__cr4__var, fp__p1__w, fp__p1__b, fp__p1__gamma, fp__p1__beta, fp__p1__mean, fp__p1__var, fp__p1_dc__w, fp__p1_dc__b, fp__p1_dc__gamma, fp__p1_dc__beta, fp__p1_dc__mean, fp__p1_dc__var, fp__p2__w, fp__p2__b, fp__p2__gamma, fp__p2__beta, fp__p2__mean, fp__p2__var, fp__p2_dc__w, fp__p2_dc__b, fp__p2_dc__gamma, fp__p2_dc__beta, fp__p2_dc__mean, fp__p2_dc__var, fp__p3__w, fp__p3__b, fp__p3__gamma, fp__p3__beta, fp__p3__mean, fp__p3__var, fp__p3_dc__w, fp__p3_dc__b, fp__p3_dc__gamma, fp__p3_dc__beta, fp__p3_dc__mean, fp__p3_dc__var, fp__p4__w, fp__p4__b, fp__p4__gamma, fp__p4__beta, fp__p4__mean, fp__p4__var, fp__p4_dc__w, fp__p4_dc__b, fp__p4_dc__gamma, fp__p4_dc__beta, fp__p4_dc__mean, fp__p4_dc__var, fp__fusion__w, fp__fusion__b, fp__fusion__gamma, fp__fusion__beta, fp__fusion__mean, fp__fusion__var, fn__cr1__w, fn__cr1__b, fn__cr1__gamma, fn__cr1__beta, fn__cr1__mean, fn__cr1__var, fn__cr2__w, fn__cr2__b, fn__cr2__gamma, fn__cr2__beta, fn__cr2__mean, fn__cr2__var, fn__cr3__w, fn__cr3__b, fn__cr3__gamma, fn__cr3__beta, fn__cr3__mean, fn__cr3__var, fn__cr4__w, fn__cr4__b, fn__cr4__gamma, fn__cr4__beta, fn__cr4__mean, fn__cr4__var, fn__p1__w, fn__p1__b, fn__p1__gamma, fn__p1__beta, fn__p1__mean, fn__p1__var, fn__p1_dc__w, fn__p1_dc__b, fn__p1_dc__gamma, fn__p1_dc__beta, fn__p1_dc__mean, fn__p1_dc__var, fn__p2__w, fn__p2__b, fn__p2__gamma, fn__p2__beta, fn__p2__mean, fn__p2__var, fn__p2_dc__w, fn__p2_dc__b, fn__p2_dc__gamma, fn__p2_dc__beta, fn__p2_dc__mean, fn__p2_dc__var, fn__p3__w, fn__p3__b, fn__p3__gamma, fn__p3__beta, fn__p3__mean, fn__p3__var, fn__p3_dc__w, fn__p3_dc__b, fn__p3_dc__gamma, fn__p3_dc__beta, fn__p3_dc__mean, fn__p3_dc__var, fn__p4__w, fn__p4__b, fn__p4__gamma, fn__p4__beta, fn__p4__mean, fn__p4__var, fn__p4_dc__w, fn__p4_dc__b, fn__p4_dc__gamma, fn__p4_dc__beta, fn__p4_dc__mean, fn__p4_dc__var, fn__fusion__w, fn__fusion__b, fn__fusion__gamma, fn__fusion__beta, fn__fusion__mean, fn__fusion__var, bn1__gamma, bn1__beta, bn1__mean, bn1__var, bn2__gamma, bn2__beta, bn2__mean, bn2__var, alpha, beta):
    raise NotImplementedError("write your pallas kernel here")


# speedup vs baseline: 3.4676x; 3.4676x over previous
import numpy as np
import jax
import jax.numpy as jnp
from jax.experimental import pallas as pl
from jax.experimental.pallas import tpu as pltpu

_BN_EPS = 1e-5
_PAR = pltpu.CompilerParams(dimension_semantics=("parallel",))


def _fold_bn(w, b, gamma, beta, mean, var):
    s = gamma * jax.lax.rsqrt(var + _BN_EPS)
    return w * s[:, None, None, None], (b - mean) * s + beta


def _prep_conv(w, b):
    cout, cin, k, _ = w.shape
    wt = jnp.transpose(w, (3, 0, 2, 1)).reshape(k, cout, k * cin)
    return wt, b.reshape(cout, 1)


def _bilin_mat(n_in, n_out):
    A = np.zeros((n_out, n_in), np.float32)
    if n_in == 1:
        A[:, 0] = 1.0
        return A
    sc = (n_in - 1) / (n_out - 1)
    for o in range(n_out):
        c = o * sc
        i0 = min(int(np.floor(c)), n_in - 1)
        i1 = min(i0 + 1, n_in - 1)
        f = c - i0
        A[o, i0] += 1.0 - f
        A[o, i1] += f
    return A


def _conv_hw(x, wkw, b, *, k, dil, W, relu=True):
    cin, HW = x.shape
    cout = wkw.shape[1]
    if k == 1:
        acc = jnp.dot(wkw[0], x, preferred_element_type=jnp.float32)
    else:
        pad = (k - 1) // 2 * dil
        ext = pad + 1
        z = jnp.zeros((cin, ext * W), x.dtype)
        xp = jnp.concatenate([z, x, z], axis=1)
        col = jax.lax.broadcasted_iota(jnp.int32, (1, HW), 1) % W
        acc = jnp.zeros((cout, HW), jnp.float32)
        for kw in range(k):
            dw = kw * dil - pad
            rows = [xp[:, (ext + kh * dil - pad) * W + dw:
                       (ext + kh * dil - pad) * W + dw + HW]
                    for kh in range(k)]
            patch = jnp.concatenate(rows, axis=0)
            part = jnp.dot(wkw[kw], patch, preferred_element_type=jnp.float32)
            if dw != 0:
                msk = ((col + dw) >= 0) & ((col + dw) < W)
                part = part * msk.astype(part.dtype)
            acc = acc + part
    acc = acc + b
    if relu:
        acc = jnp.maximum(acc, 0.0)
    return acc


def _front_kernel(y_ref, w_ref, b_ref, mt_ref, imap_ref, up_ref, m_ref, *, w_in):
    up_small = _conv_hw(y_ref[...], w_ref[...], b_ref[...], k=7, dil=1, W=w_in)
    src = jnp.concatenate([up_small, imap_ref[...]], axis=0)
    big = jnp.dot(src, mt_ref[...], preferred_element_type=jnp.float32)
    c1 = up_ref.shape[0]
    up_ref[...] = big[:c1]
    m_ref[...] = jax.nn.sigmoid(big[c1:c1 + 1])


def _ceb_kernel(x_ref, m_ref, crw_ref, crb_ref,
                p1w, p1b, d1w, d1b, p2w, p2b, d2w, d2b,
                p3w, p3b, d3w, d3b, p4w, p4b, d4w, d4b,
                fw, fb, o_ref, *, W, cs):
    sel = pl.program_id(0) % 2
    m = m_ref[...]
    gate = jnp.where(sel == 0, m, 1.0 - m)
    feat = x_ref[...] * gate
    cr = jnp.maximum(
        jnp.dot(crw_ref[...], feat, preferred_element_type=jnp.float32)
        + crb_ref[...], 0.0)
    p1 = _conv_hw(cr[0 * cs:1 * cs], p1w[...], p1b[...], k=1, dil=1, W=W)
    d1 = _conv_hw(p1, d1w[...], d1b[...], k=3, dil=1, W=W)
    p2 = _conv_hw(cr[1 * cs:2 * cs] + d1, p2w[...], p2b[...], k=3, dil=1, W=W)
    d2 = _conv_hw(p2, d2w[...], d2b[...], k=3, dil=2, W=W)
    p3 = _conv_hw(cr[2 * cs:3 * cs] + d2, p3w[...], p3b[...], k=5, dil=1, W=W)
    d3 = _conv_hw(p3, d3w[...], d3b[...], k=3, dil=4, W=W)
    p4 = _conv_hw(cr[3 * cs:4 * cs] + d3, p4w[...], p4b[...], k=7, dil=1, W=W)
    d4 = _conv_hw(p4, d4w[...], d4b[...], k=3, dil=8, W=W)
    cat = jnp.concatenate([d1, d2, d3, d4], axis=0)
    o_ref[...] = _conv_hw(cat, fw[...], fb[...], k=1, dil=1, W=W)


def _back_kernel(up_ref, fp_ref, fn_ref, rp_ref, wz_ref, ob_ref,
                 r2_ref, om_ref, *, W):
    rp = rp_ref[...]
    r1 = jnp.maximum(rp[0] * up_ref[...] + rp[1] * fp_ref[...] + rp[2], 0.0)
    r2 = jnp.maximum(rp[3] * r1 + rp[4] * fn_ref[...] + rp[5], 0.0)
    r2_ref[...] = r2
    c1, HW = r2.shape
    ext = 4
    z = jnp.zeros((c1, ext * W), r2.dtype)
    r2p = jnp.concatenate([z, r2, z], axis=1)
    Z = jnp.dot(wz_ref[...], r2p, preferred_element_type=jnp.float32)
    col = jax.lax.broadcasted_iota(jnp.int32, (1, HW), 1) % W
    acc = jnp.zeros((1, HW), jnp.float32) + ob_ref[0, 0]
    for kh in range(7):
        for kw in range(7):
            t = kh * 7 + kw
            dw = kw - 3
            s = (ext + kh - 3) * W + dw
            part = Z[t:t + 1, s:s + HW]
            if dw != 0:
                msk = ((col + dw) >= 0) & ((col + dw) < W)
                part = part * msk.astype(part.dtype)
            acc = acc + part
    om_ref[...] = acc


def kernel(x, y, in_map, up__w, up__b, up__gamma, up__beta, up__mean, up__var, up2__w, up2__b, up2__gamma, up2__beta, up2__mean, up2__var, output_map__w, output_map__b, fp__cr1__w, fp__cr1__b, fp__cr1__gamma, fp__cr1__beta, fp__cr1__mean, fp__cr1__var, fp__cr2__w, fp__cr2__b, fp__cr2__gamma, fp__cr2__beta, fp__cr2__mean, fp__cr2__var, fp__cr3__w, fp__cr3__b, fp__cr3__gamma, fp__cr3__beta, fp__cr3__mean, fp__cr3__var, fp__cr4__w, fp__cr4__b, fp__cr4__gamma, fp__cr4__beta, fp__cr4__mean, fp__cr4__var, fp__p1__w, fp__p1__b, fp__p1__gamma, fp__p1__beta, fp__p1__mean, fp__p1__var, fp__p1_dc__w, fp__p1_dc__b, fp__p1_dc__gamma, fp__p1_dc__beta, fp__p1_dc__mean, fp__p1_dc__var, fp__p2__w, fp__p2__b, fp__p2__gamma, fp__p2__beta, fp__p2__mean, fp__p2__var, fp__p2_dc__w, fp__p2_dc__b, fp__p2_dc__gamma, fp__p2_dc__beta, fp__p2_dc__mean, fp__p2_dc__var, fp__p3__w, fp__p3__b, fp__p3__gamma, fp__p3__beta, fp__p3__mean, fp__p3__var, fp__p3_dc__w, fp__p3_dc__b, fp__p3_dc__gamma, fp__p3_dc__beta, fp__p3_dc__mean, fp__p3_dc__var, fp__p4__w, fp__p4__b, fp__p4__gamma, fp__p4__beta, fp__p4__mean, fp__p4__var, fp__p4_dc__w, fp__p4_dc__b, fp__p4_dc__gamma, fp__p4_dc__beta, fp__p4_dc__mean, fp__p4_dc__var, fp__fusion__w, fp__fusion__b, fp__fusion__gamma, fp__fusion__beta, fp__fusion__mean, fp__fusion__var, fn__cr1__w, fn__cr1__b, fn__cr1__gamma, fn__cr1__beta, fn__cr1__mean, fn__cr1__var, fn__cr2__w, fn__cr2__b, fn__cr2__gamma, fn__cr2__beta, fn__cr2__mean, fn__cr2__var, fn__cr3__w, fn__cr3__b, fn__cr3__gamma, fn__cr3__beta, fn__cr3__mean, fn__cr3__var, fn__cr4__w, fn__cr4__b, fn__cr4__gamma, fn__cr4__beta, fn__cr4__mean, fn__cr4__var, fn__p1__w, fn__p1__b, fn__p1__gamma, fn__p1__beta, fn__p1__mean, fn__p1__var, fn__p1_dc__w, fn__p1_dc__b, fn__p1_dc__gamma, fn__p1_dc__beta, fn__p1_dc__mean, fn__p1_dc__var, fn__p2__w, fn__p2__b, fn__p2__gamma, fn__p2__beta, fn__p2__mean, fn__p2__var, fn__p2_dc__w, fn__p2_dc__b, fn__p2_dc__gamma, fn__p2_dc__beta, fn__p2_dc__mean, fn__p2_dc__var, fn__p3__w, fn__p3__b, fn__p3__gamma, fn__p3__beta, fn__p3__mean, fn__p3__var, fn__p3_dc__w, fn__p3_dc__b, fn__p3_dc__gamma, fn__p3_dc__beta, fn__p3_dc__mean, fn__p3_dc__var, fn__p4__w, fn__p4__b, fn__p4__gamma, fn__p4__beta, fn__p4__mean, fn__p4__var, fn__p4_dc__w, fn__p4_dc__b, fn__p4_dc__gamma, fn__p4_dc__beta, fn__p4_dc__mean, fn__p4_dc__var, fn__fusion__w, fn__fusion__b, fn__fusion__gamma, fn__fusion__beta, fn__fusion__mean, fn__fusion__var, bn1__gamma, bn1__beta, bn1__mean, bn1__var, bn2__gamma, bn2__beta, bn2__mean, bn2__var, alpha, beta):
    import functools

    N, C1, H, W = x.shape
    C2 = y.shape[1]
    h, w = H // 2, W // 2
    HW, hw = H * W, h * w
    cs = C1 // 4

    wup, bup = _prep_conv(*_fold_bn(up__w, up__b, up__gamma, up__beta,
                                    up__mean, up__var))
    MT = jnp.asarray(np.kron(_bilin_mat(h, H), _bilin_mat(w, W)).T)
    y2 = y.reshape(N, C2, hw)
    imap2 = in_map.reshape(N, 1, hw)

    up_flat, m_flat = pl.pallas_call(
        functools.partial(_front_kernel, w_in=w),
        out_shape=(jax.ShapeDtypeStruct((N, C1, HW), jnp.float32),
                   jax.ShapeDtypeStruct((N, 1, HW), jnp.float32)),
        grid=(N,),
        in_specs=[pl.BlockSpec((None, C2, hw), lambda n: (n, 0, 0)),
                  pl.BlockSpec(wup.shape, lambda n: (0, 0, 0)),
                  pl.BlockSpec((C1, 1), lambda n: (0, 0)),
                  pl.BlockSpec((hw, HW), lambda n: (0, 0)),
                  pl.BlockSpec((None, 1, hw), lambda n: (n, 0, 0))],
        out_specs=(pl.BlockSpec((None, C1, HW), lambda n: (n, 0, 0)),
                   pl.BlockSpec((None, 1, HW), lambda n: (n, 0, 0))),
        compiler_params=_PAR,
    )(y2, wup, bup, MT, imap2)

    branches = (
        dict(cr1=(fp__cr1__w, fp__cr1__b, fp__cr1__gamma, fp__cr1__beta, fp__cr1__mean, fp__cr1__var),
             cr2=(fp__cr2__w, fp__cr2__b, fp__cr2__gamma, fp__cr2__beta, fp__cr2__mean, fp__cr2__var),
             cr3=(fp__cr3__w, fp__cr3__b, fp__cr3__gamma, fp__cr3__beta, fp__cr3__mean, fp__cr3__var),
             cr4=(fp__cr4__w, fp__cr4__b, fp__cr4__gamma, fp__cr4__beta, fp__cr4__mean, fp__cr4__var),
             p1=(fp__p1__w, fp__p1__b, fp__p1__gamma, fp__p1__beta, fp__p1__mean, fp__p1__var),
             p1_dc=(fp__p1_dc__w, fp__p1_dc__b, fp__p1_dc__gamma, fp__p1_dc__beta, fp__p1_dc__mean, fp__p1_dc__var),
             p2=(fp__p2__w, fp__p2__b, fp__p2__gamma, fp__p2__beta, fp__p2__mean, fp__p2__var),
             p2_dc=(fp__p2_dc__w, fp__p2_dc__b, fp__p2_dc__gamma, fp__p2_dc__beta, fp__p2_dc__mean, fp__p2_dc__var),
             p3=(fp__p3__w, fp__p3__b, fp__p3__gamma, fp__p3__beta, fp__p3__mean, fp__p3__var),
             p3_dc=(fp__p3_dc__w, fp__p3_dc__b, fp__p3_dc__gamma, fp__p3_dc__beta, fp__p3_dc__mean, fp__p3_dc__var),
             p4=(fp__p4__w, fp__p4__b, fp__p4__gamma, fp__p4__beta, fp__p4__mean, fp__p4__var),
             p4_dc=(fp__p4_dc__w, fp__p4_dc__b, fp__p4_dc__gamma, fp__p4_dc__beta, fp__p4_dc__mean, fp__p4_dc__var),
             fusion=(fp__fusion__w, fp__fusion__b, fp__fusion__gamma, fp__fusion__beta, fp__fusion__mean, fp__fusion__var)),
        dict(cr1=(fn__cr1__w, fn__cr1__b, fn__cr1__gamma, fn__cr1__beta, fn__cr1__mean, fn__cr1__var),
             cr2=(fn__cr2__w, fn__cr2__b, fn__cr2__gamma, fn__cr2__beta, fn__cr2__mean, fn__cr2__var),
             cr3=(fn__cr3__w, fn__cr3__b, fn__cr3__gamma, fn__cr3__beta, fn__cr3__mean, fn__cr3__var),
             cr4=(fn__cr4__w, fn__cr4__b, fn__cr4__gamma, fn__cr4__beta, fn__cr4__mean, fn__cr4__var),
             p1=(fn__p1__w, fn__p1__b, fn__p1__gamma, fn__p1__beta, fn__p1__mean, fn__p1__var),
             p1_dc=(fn__p1_dc__w, fn__p1_dc__b, fn__p1_dc__gamma, fn__p1_dc__beta, fn__p1_dc__mean, fn__p1_dc__var),
             p2=(fn__p2__w, fn__p2__b, fn__p2__gamma, fn__p2__beta, fn__p2__mean, fn__p2__var),
             p2_dc=(fn__p2_dc__w, fn__p2_dc__b, fn__p2_dc__gamma, fn__p2_dc__beta, fn__p2_dc__mean, fn__p2_dc__var),
             p3=(fn__p3__w, fn__p3__b, fn__p3__gamma, fn__p3__beta, fn__p3__mean, fn__p3__var),
             p3_dc=(fn__p3_dc__w, fn__p3_dc__b, fn__p3_dc__gamma, fn__p3_dc__beta, fn__p3_dc__mean, fn__p3_dc__var),
             p4=(fn__p4__w, fn__p4__b, fn__p4__gamma, fn__p4__beta, fn__p4__mean, fn__p4__var),
             p4_dc=(fn__p4_dc__w, fn__p4_dc__b, fn__p4_dc__gamma, fn__p4_dc__beta, fn__p4_dc__mean, fn__p4_dc__var),
             fusion=(fn__fusion__w, fn__fusion__b, fn__fusion__gamma, fn__fusion__beta, fn__fusion__mean, fn__fusion__var)))

    def folded(br, nm):
        return _fold_bn(*br[nm])

    crw, crb = [], []
    for br in branches:
        ws, bs = [], []
        for nm in ("cr1", "cr2", "cr3", "cr4"):
            wf, bf = folded(br, nm)
            ws.append(wf.reshape(cs, C1))
            bs.append(bf)
        crw.append(jnp.concatenate(ws, axis=0))
        crb.append(jnp.concatenate(bs, axis=0).reshape(C1, 1))
    crw = jnp.stack(crw)
    crb = jnp.stack(crb)

    def stacked(nm):
        pw, pb = _prep_conv(*folded(branches[0], nm))
        nw, nb = _prep_conv(*folded(branches[1], nm))
        return jnp.stack([pw, nw]), jnp.stack([pb, nb])

    conv_names = ("p1", "p1_dc", "p2", "p2_dc", "p3", "p3_dc",
                  "p4", "p4_dc", "fusion")
    packed = [a for nm in conv_names for a in stacked(nm)]

    x2 = x.reshape(N, C1, HW)
    wspecs = []
    for arr in packed:
        bs = (None,) + arr.shape[1:]
        wspecs.append(
            pl.BlockSpec(bs, lambda i, nz=len(bs) - 1: (i % 2,) + (0,) * nz))

    ceb_out = pl.pallas_call(
        functools.partial(_ceb_kernel, W=W, cs=cs),
        out_shape=jax.ShapeDtypeStruct((2, N, C1, HW), jnp.float32),
        grid=(2 * N,),
        in_specs=[pl.BlockSpec((None, C1, HW), lambda i: (i // 2, 0, 0)),
                  pl.BlockSpec((None, 1, HW), lambda i: (i // 2, 0, 0)),
                  pl.BlockSpec((None, C1, C1), lambda i: (i % 2, 0, 0)),
                  pl.BlockSpec((None, C1, 1), lambda i: (i % 2, 0, 0))] + wspecs,
        out_specs=pl.BlockSpec((None, None, C1, HW),
                               lambda i: (i % 2, i // 2, 0, 0)),
        compiler_params=_PAR,
    )(x2, m_flat, crw, crb, *packed)

    s1 = bn1__gamma * jax.lax.rsqrt(bn1__var + _BN_EPS)
    b1 = bn1__beta - bn1__mean * s1
    s2 = bn2__gamma * jax.lax.rsqrt(bn2__var + _BN_EPS)
    b2 = bn2__beta - bn2__mean * s2
    rparams = jnp.stack([s1, -alpha[0] * s1, b1,
                         s2, beta[0] * s2, b2]).reshape(6, C1, 1)
    wz = jnp.transpose(output_map__w[0], (1, 2, 0)).reshape(49, C1)
    ob = output_map__b.reshape(1, 1)

    r2_flat, om_flat = pl.pallas_call(
        functools.partial(_back_kernel, W=W),
        out_shape=(jax.ShapeDtypeStruct((N, C1, HW), jnp.float32),
                   jax.ShapeDtypeStruct((N, 1, HW), jnp.float32)),
        grid=(N,),
        in_specs=[pl.BlockSpec((None, C1, HW), lambda n: (n, 0, 0)),
                  pl.BlockSpec((None, None, C1, HW), lambda n: (0, n, 0, 0)),
                  pl.BlockSpec((None, None, C1, HW), lambda n: (1, n, 0, 0)),
                  pl.BlockSpec((6, C1, 1), lambda n: (0, 0, 0)),
                  pl.BlockSpec((49, C1), lambda n: (0, 0)),
                  pl.BlockSpec((1, 1), lambda n: (0, 0))],
        out_specs=(pl.BlockSpec((None, C1, HW), lambda n: (n, 0, 0)),
                   pl.BlockSpec((None, 1, HW), lambda n: (n, 0, 0))),
        compiler_params=_PAR,
    )(up_flat, ceb_out, ceb_out, rparams, wz, ob)

    return r2_flat.reshape(N, C1, H, W), om_flat.reshape(N, 1, H, W)


# speedup vs baseline: 3.6064x; 1.0401x over previous
import numpy as np
import jax
import jax.numpy as jnp
from jax.experimental import pallas as pl
from jax.experimental.pallas import tpu as pltpu

_BN_EPS = 1e-5
_PAR = pltpu.CompilerParams(dimension_semantics=("parallel",))


def _fold_bn(w, b, gamma, beta, mean, var):
    s = gamma * jax.lax.rsqrt(var + _BN_EPS)
    return w * s[:, None, None, None], (b - mean) * s + beta


def _prep_conv(w, b):
    cout, cin, k, _ = w.shape
    wt = jnp.transpose(w, (3, 0, 2, 1)).reshape(k, cout, k * cin)
    return wt.astype(jnp.bfloat16), b.reshape(cout, 1)


def _bilin_mat(n_in, n_out):
    A = np.zeros((n_out, n_in), np.float32)
    if n_in == 1:
        A[:, 0] = 1.0
        return A
    sc = (n_in - 1) / (n_out - 1)
    for o in range(n_out):
        c = o * sc
        i0 = min(int(np.floor(c)), n_in - 1)
        i1 = min(i0 + 1, n_in - 1)
        f = c - i0
        A[o, i0] += 1.0 - f
        A[o, i1] += f
    return A


def _conv_hw(x, wkw, b, *, k, dil, W, relu=True):
    x = x.astype(jnp.bfloat16)
    cin, HW = x.shape
    cout = wkw.shape[1]
    if k == 1:
        acc = jnp.dot(wkw[0], x, preferred_element_type=jnp.float32)
    else:
        pad = (k - 1) // 2 * dil
        ext = pad + 1
        z = jnp.zeros((cin, ext * W), x.dtype)
        xp = jnp.concatenate([z, x, z], axis=1)
        col = jax.lax.broadcasted_iota(jnp.int32, (1, HW), 1) % W
        acc = jnp.zeros((cout, HW), jnp.float32)
        for kw in range(k):
            dw = kw * dil - pad
            rows = [xp[:, (ext + kh * dil - pad) * W + dw:
                       (ext + kh * dil - pad) * W + dw + HW]
                    for kh in range(k)]
            patch = jnp.concatenate(rows, axis=0)
            part = jnp.dot(wkw[kw], patch, preferred_element_type=jnp.float32)
            if dw != 0:
                msk = ((col + dw) >= 0) & ((col + dw) < W)
                part = part * msk.astype(part.dtype)
            acc = acc + part
    acc = acc + b
    if relu:
        acc = jnp.maximum(acc, 0.0)
    return acc


def _front_kernel(y_ref, w_ref, b_ref, mt_ref, imap_ref, up_ref, m_ref, *, w_in):
    up_small = _conv_hw(y_ref[...], w_ref[...], b_ref[...], k=7, dil=1, W=w_in)
    src = jnp.concatenate([up_small, imap_ref[...]], axis=0)
    big = jnp.dot(src.astype(jnp.bfloat16), mt_ref[...],
                  preferred_element_type=jnp.float32)
    c1 = up_ref.shape[0]
    up_ref[...] = big[:c1]
    m_ref[...] = jax.nn.sigmoid(big[c1:c1 + 1])


def _ceb_kernel(x_ref, m_ref, crw_ref, crb_ref,
                p1w, p1b, d1w, d1b, p2w, p2b, d2w, d2b,
                p3w, p3b, d3w, d3b, p4w, p4b, d4w, d4b,
                fw, fb, o_ref, *, W, cs):
    sel = pl.program_id(0) % 2
    m = m_ref[...]
    gate = jnp.where(sel == 0, m, 1.0 - m)
    feat = x_ref[...] * gate
    cr = jnp.maximum(
        jnp.dot(crw_ref[...], feat.astype(jnp.bfloat16),
                preferred_element_type=jnp.float32)
        + crb_ref[...], 0.0)
    p1 = _conv_hw(cr[0 * cs:1 * cs], p1w[...], p1b[...], k=1, dil=1, W=W)
    d1 = _conv_hw(p1, d1w[...], d1b[...], k=3, dil=1, W=W)
    p2 = _conv_hw(cr[1 * cs:2 * cs] + d1, p2w[...], p2b[...], k=3, dil=1, W=W)
    d2 = _conv_hw(p2, d2w[...], d2b[...], k=3, dil=2, W=W)
    p3 = _conv_hw(cr[2 * cs:3 * cs] + d2, p3w[...], p3b[...], k=5, dil=1, W=W)
    d3 = _conv_hw(p3, d3w[...], d3b[...], k=3, dil=4, W=W)
    p4 = _conv_hw(cr[3 * cs:4 * cs] + d3, p4w[...], p4b[...], k=7, dil=1, W=W)
    d4 = _conv_hw(p4, d4w[...], d4b[...], k=3, dil=8, W=W)
    cat = jnp.concatenate([d1, d2, d3, d4], axis=0)
    o_ref[...] = _conv_hw(cat, fw[...], fb[...], k=1, dil=1, W=W)


def _back_kernel(up_ref, fp_ref, fn_ref, rp_ref, wz_ref, ob_ref,
                 r2_ref, om_ref, *, W):
    rp = rp_ref[...]
    r1 = jnp.maximum(rp[0] * up_ref[...] + rp[1] * fp_ref[...] + rp[2], 0.0)
    r2 = jnp.maximum(rp[3] * r1 + rp[4] * fn_ref[...] + rp[5], 0.0)
    r2_ref[...] = r2
    c1, HW = r2.shape
    ext = 4
    r2h = r2.astype(jnp.bfloat16)
    z = jnp.zeros((c1, ext * W), r2h.dtype)
    r2p = jnp.concatenate([z, r2h, z], axis=1)
    Z = jnp.dot(wz_ref[...], r2p, preferred_element_type=jnp.float32)
    col = jax.lax.broadcasted_iota(jnp.int32, (1, HW), 1) % W
    acc = jnp.zeros((1, HW), jnp.float32) + ob_ref[0, 0]
    for kh in range(7):
        for kw in range(7):
            t = kh * 7 + kw
            dw = kw - 3
            s = (ext + kh - 3) * W + dw
            part = Z[t:t + 1, s:s + HW]
            if dw != 0:
                msk = ((col + dw) >= 0) & ((col + dw) < W)
                part = part * msk.astype(part.dtype)
            acc = acc + part
    om_ref[...] = acc


def kernel(x, y, in_map, up__w, up__b, up__gamma, up__beta, up__mean, up__var, up2__w, up2__b, up2__gamma, up2__beta, up2__mean, up2__var, output_map__w, output_map__b, fp__cr1__w, fp__cr1__b, fp__cr1__gamma, fp__cr1__beta, fp__cr1__mean, fp__cr1__var, fp__cr2__w, fp__cr2__b, fp__cr2__gamma, fp__cr2__beta, fp__cr2__mean, fp__cr2__var, fp__cr3__w, fp__cr3__b, fp__cr3__gamma, fp__cr3__beta, fp__cr3__mean, fp__cr3__var, fp__cr4__w, fp__cr4__b, fp__cr4__gamma, fp__cr4__beta, fp__cr4__mean, fp__cr4__var, fp__p1__w, fp__p1__b, fp__p1__gamma, fp__p1__beta, fp__p1__mean, fp__p1__var, fp__p1_dc__w, fp__p1_dc__b, fp__p1_dc__gamma, fp__p1_dc__beta, fp__p1_dc__mean, fp__p1_dc__var, fp__p2__w, fp__p2__b, fp__p2__gamma, fp__p2__beta, fp__p2__mean, fp__p2__var, fp__p2_dc__w, fp__p2_dc__b, fp__p2_dc__gamma, fp__p2_dc__beta, fp__p2_dc__mean, fp__p2_dc__var, fp__p3__w, fp__p3__b, fp__p3__gamma, fp__p3__beta, fp__p3__mean, fp__p3__var, fp__p3_dc__w, fp__p3_dc__b, fp__p3_dc__gamma, fp__p3_dc__beta, fp__p3_dc__mean, fp__p3_dc__var, fp__p4__w, fp__p4__b, fp__p4__gamma, fp__p4__beta, fp__p4__mean, fp__p4__var, fp__p4_dc__w, fp__p4_dc__b, fp__p4_dc__gamma, fp__p4_dc__beta, fp__p4_dc__mean, fp__p4_dc__var, fp__fusion__w, fp__fusion__b, fp__fusion__gamma, fp__fusion__beta, fp__fusion__mean, fp__fusion__var, fn__cr1__w, fn__cr1__b, fn__cr1__gamma, fn__cr1__beta, fn__cr1__mean, fn__cr1__var, fn__cr2__w, fn__cr2__b, fn__cr2__gamma, fn__cr2__beta, fn__cr2__mean, fn__cr2__var, fn__cr3__w, fn__cr3__b, fn__cr3__gamma, fn__cr3__beta, fn__cr3__mean, fn__cr3__var, fn__cr4__w, fn__cr4__b, fn__cr4__gamma, fn__cr4__beta, fn__cr4__mean, fn__cr4__var, fn__p1__w, fn__p1__b, fn__p1__gamma, fn__p1__beta, fn__p1__mean, fn__p1__var, fn__p1_dc__w, fn__p1_dc__b, fn__p1_dc__gamma, fn__p1_dc__beta, fn__p1_dc__mean, fn__p1_dc__var, fn__p2__w, fn__p2__b, fn__p2__gamma, fn__p2__beta, fn__p2__mean, fn__p2__var, fn__p2_dc__w, fn__p2_dc__b, fn__p2_dc__gamma, fn__p2_dc__beta, fn__p2_dc__mean, fn__p2_dc__var, fn__p3__w, fn__p3__b, fn__p3__gamma, fn__p3__beta, fn__p3__mean, fn__p3__var, fn__p3_dc__w, fn__p3_dc__b, fn__p3_dc__gamma, fn__p3_dc__beta, fn__p3_dc__mean, fn__p3_dc__var, fn__p4__w, fn__p4__b, fn__p4__gamma, fn__p4__beta, fn__p4__mean, fn__p4__var, fn__p4_dc__w, fn__p4_dc__b, fn__p4_dc__gamma, fn__p4_dc__beta, fn__p4_dc__mean, fn__p4_dc__var, fn__fusion__w, fn__fusion__b, fn__fusion__gamma, fn__fusion__beta, fn__fusion__mean, fn__fusion__var, bn1__gamma, bn1__beta, bn1__mean, bn1__var, bn2__gamma, bn2__beta, bn2__mean, bn2__var, alpha, beta):
    import functools

    N, C1, H, W = x.shape
    C2 = y.shape[1]
    h, w = H // 2, W // 2
    HW, hw = H * W, h * w
    cs = C1 // 4

    wup, bup = _prep_conv(*_fold_bn(up__w, up__b, up__gamma, up__beta,
                                    up__mean, up__var))
    MT = jnp.asarray(np.kron(_bilin_mat(h, H), _bilin_mat(w, W)).T
                     ).astype(jnp.bfloat16)
    y2 = y.reshape(N, C2, hw)
    imap2 = in_map.reshape(N, 1, hw)

    up_flat, m_flat = pl.pallas_call(
        functools.partial(_front_kernel, w_in=w),
        out_shape=(jax.ShapeDtypeStruct((N, C1, HW), jnp.float32),
                   jax.ShapeDtypeStruct((N, 1, HW), jnp.float32)),
        grid=(N,),
        in_specs=[pl.BlockSpec((None, C2, hw), lambda n: (n, 0, 0)),
                  pl.BlockSpec(wup.shape, lambda n: (0, 0, 0)),
                  pl.BlockSpec((C1, 1), lambda n: (0, 0)),
                  pl.BlockSpec((hw, HW), lambda n: (0, 0)),
                  pl.BlockSpec((None, 1, hw), lambda n: (n, 0, 0))],
        out_specs=(pl.BlockSpec((None, C1, HW), lambda n: (n, 0, 0)),
                   pl.BlockSpec((None, 1, HW), lambda n: (n, 0, 0))),
        compiler_params=_PAR,
    )(y2, wup, bup, MT, imap2)

    branches = (
        dict(cr1=(fp__cr1__w, fp__cr1__b, fp__cr1__gamma, fp__cr1__beta, fp__cr1__mean, fp__cr1__var),
             cr2=(fp__cr2__w, fp__cr2__b, fp__cr2__gamma, fp__cr2__beta, fp__cr2__mean, fp__cr2__var),
             cr3=(fp__cr3__w, fp__cr3__b, fp__cr3__gamma, fp__cr3__beta, fp__cr3__mean, fp__cr3__var),
             cr4=(fp__cr4__w, fp__cr4__b, fp__cr4__gamma, fp__cr4__beta, fp__cr4__mean, fp__cr4__var),
             p1=(fp__p1__w, fp__p1__b, fp__p1__gamma, fp__p1__beta, fp__p1__mean, fp__p1__var),
             p1_dc=(fp__p1_dc__w, fp__p1_dc__b, fp__p1_dc__gamma, fp__p1_dc__beta, fp__p1_dc__mean, fp__p1_dc__var),
             p2=(fp__p2__w, fp__p2__b, fp__p2__gamma, fp__p2__beta, fp__p2__mean, fp__p2__var),
             p2_dc=(fp__p2_dc__w, fp__p2_dc__b, fp__p2_dc__gamma, fp__p2_dc__beta, fp__p2_dc__mean, fp__p2_dc__var),
             p3=(fp__p3__w, fp__p3__b, fp__p3__gamma, fp__p3__beta, fp__p3__mean, fp__p3__var),
             p3_dc=(fp__p3_dc__w, fp__p3_dc__b, fp__p3_dc__gamma, fp__p3_dc__beta, fp__p3_dc__mean, fp__p3_dc__var),
             p4=(fp__p4__w, fp__p4__b, fp__p4__gamma, fp__p4__beta, fp__p4__mean, fp__p4__var),
             p4_dc=(fp__p4_dc__w, fp__p4_dc__b, fp__p4_dc__gamma, fp__p4_dc__beta, fp__p4_dc__mean, fp__p4_dc__var),
             fusion=(fp__fusion__w, fp__fusion__b, fp__fusion__gamma, fp__fusion__beta, fp__fusion__mean, fp__fusion__var)),
        dict(cr1=(fn__cr1__w, fn__cr1__b, fn__cr1__gamma, fn__cr1__beta, fn__cr1__mean, fn__cr1__var),
             cr2=(fn__cr2__w, fn__cr2__b, fn__cr2__gamma, fn__cr2__beta, fn__cr2__mean, fn__cr2__var),
             cr3=(fn__cr3__w, fn__cr3__b, fn__cr3__gamma, fn__cr3__beta, fn__cr3__mean, fn__cr3__var),
             cr4=(fn__cr4__w, fn__cr4__b, fn__cr4__gamma, fn__cr4__beta, fn__cr4__mean, fn__cr4__var),
             p1=(fn__p1__w, fn__p1__b, fn__p1__gamma, fn__p1__beta, fn__p1__mean, fn__p1__var),
             p1_dc=(fn__p1_dc__w, fn__p1_dc__b, fn__p1_dc__gamma, fn__p1_dc__beta, fn__p1_dc__mean, fn__p1_dc__var),
             p2=(fn__p2__w, fn__p2__b, fn__p2__gamma, fn__p2__beta, fn__p2__mean, fn__p2__var),
             p2_dc=(fn__p2_dc__w, fn__p2_dc__b, fn__p2_dc__gamma, fn__p2_dc__beta, fn__p2_dc__mean, fn__p2_dc__var),
             p3=(fn__p3__w, fn__p3__b, fn__p3__gamma, fn__p3__beta, fn__p3__mean, fn__p3__var),
             p3_dc=(fn__p3_dc__w, fn__p3_dc__b, fn__p3_dc__gamma, fn__p3_dc__beta, fn__p3_dc__mean, fn__p3_dc__var),
             p4=(fn__p4__w, fn__p4__b, fn__p4__gamma, fn__p4__beta, fn__p4__mean, fn__p4__var),
             p4_dc=(fn__p4_dc__w, fn__p4_dc__b, fn__p4_dc__gamma, fn__p4_dc__beta, fn__p4_dc__mean, fn__p4_dc__var),
             fusion=(fn__fusion__w, fn__fusion__b, fn__fusion__gamma, fn__fusion__beta, fn__fusion__mean, fn__fusion__var)))

    def folded(br, nm):
        return _fold_bn(*br[nm])

    crw, crb = [], []
    for br in branches:
        ws, bs = [], []
        for nm in ("cr1", "cr2", "cr3", "cr4"):
            wf, bf = folded(br, nm)
            ws.append(wf.reshape(cs, C1))
            bs.append(bf)
        crw.append(jnp.concatenate(ws, axis=0))
        crb.append(jnp.concatenate(bs, axis=0).reshape(C1, 1))
    crw = jnp.stack(crw).astype(jnp.bfloat16)
    crb = jnp.stack(crb)

    def stacked(nm):
        pw, pb = _prep_conv(*folded(branches[0], nm))
        nw, nb = _prep_conv(*folded(branches[1], nm))
        return jnp.stack([pw, nw]), jnp.stack([pb, nb])

    conv_names = ("p1", "p1_dc", "p2", "p2_dc", "p3", "p3_dc",
                  "p4", "p4_dc", "fusion")
    packed = [a for nm in conv_names for a in stacked(nm)]

    x2 = x.reshape(N, C1, HW)
    wspecs = []
    for arr in packed:
        bs = (None,) + arr.shape[1:]
        wspecs.append(
            pl.BlockSpec(bs, lambda i, nz=len(bs) - 1: (i % 2,) + (0,) * nz))

    ceb_out = pl.pallas_call(
        functools.partial(_ceb_kernel, W=W, cs=cs),
        out_shape=jax.ShapeDtypeStruct((2, N, C1, HW), jnp.float32),
        grid=(2 * N,),
        in_specs=[pl.BlockSpec((None, C1, HW), lambda i: (i // 2, 0, 0)),
                  pl.BlockSpec((None, 1, HW), lambda i: (i // 2, 0, 0)),
                  pl.BlockSpec((None, C1, C1), lambda i: (i % 2, 0, 0)),
                  pl.BlockSpec((None, C1, 1), lambda i: (i % 2, 0, 0))] + wspecs,
        out_specs=pl.BlockSpec((None, None, C1, HW),
                               lambda i: (i % 2, i // 2, 0, 0)),
        compiler_params=_PAR,
    )(x2, m_flat, crw, crb, *packed)

    s1 = bn1__gamma * jax.lax.rsqrt(bn1__var + _BN_EPS)
    b1 = bn1__beta - bn1__mean * s1
    s2 = bn2__gamma * jax.lax.rsqrt(bn2__var + _BN_EPS)
    b2 = bn2__beta - bn2__mean * s2
    rparams = jnp.stack([s1, -alpha[0] * s1, b1,
                         s2, beta[0] * s2, b2]).reshape(6, C1, 1)
    wz = jnp.transpose(output_map__w[0], (1, 2, 0)).reshape(49, C1)
    wz = wz.astype(jnp.bfloat16)
    ob = output_map__b.reshape(1, 1)

    r2_flat, om_flat = pl.pallas_call(
        functools.partial(_back_kernel, W=W),
        out_shape=(jax.ShapeDtypeStruct((N, C1, HW), jnp.float32),
                   jax.ShapeDtypeStruct((N, 1, HW), jnp.float32)),
        grid=(N,),
        in_specs=[pl.BlockSpec((None, C1, HW), lambda n: (n, 0, 0)),
                  pl.BlockSpec((None, None, C1, HW), lambda n: (0, n, 0, 0)),
                  pl.BlockSpec((None, None, C1, HW), lambda n: (1, n, 0, 0)),
                  pl.BlockSpec((6, C1, 1), lambda n: (0, 0, 0)),
                  pl.BlockSpec((49, C1), lambda n: (0, 0)),
                  pl.BlockSpec((1, 1), lambda n: (0, 0))],
        out_specs=(pl.BlockSpec((None, C1, HW), lambda n: (n, 0, 0)),
                   pl.BlockSpec((None, 1, HW), lambda n: (n, 0, 0))),
        compiler_params=_PAR,
    )(up_flat, ceb_out, ceb_out, rparams, wz, ob)

    return r2_flat.reshape(N, C1, H, W), om_flat.reshape(N, 1, H, W)


# speedup vs baseline: 3.6145x; 1.0022x over previous
import functools

import numpy as np
import jax
import jax.numpy as jnp
from jax.experimental import pallas as pl
from jax.experimental.pallas import tpu as pltpu

_BN_EPS = 1e-5
_PAR = pltpu.CompilerParams(dimension_semantics=("parallel",))


def _fold_bn(w, b, gamma, beta, mean, var):
    s = gamma * jax.lax.rsqrt(var + _BN_EPS)
    return w * s[:, None, None, None], (b - mean) * s + beta


def _prep_conv(w, b):
    cout, cin, k, _ = w.shape
    wt = jnp.transpose(w, (3, 0, 2, 1)).reshape(k, cout, k * cin)
    return wt.astype(jnp.bfloat16), b.reshape(cout, 1)


def _bilin_mat(n_in, n_out):
    A = np.zeros((n_out, n_in), np.float32)
    if n_in == 1:
        A[:, 0] = 1.0
        return A
    sc = (n_in - 1) / (n_out - 1)
    for o in range(n_out):
        c = o * sc
        i0 = min(int(np.floor(c)), n_in - 1)
        i1 = min(i0 + 1, n_in - 1)
        f = c - i0
        A[o, i0] += 1.0 - f
        A[o, i1] += f
    return A


def _conv_hw(x, wkw, b, *, k, dil, W, relu=True):
    x = x.astype(jnp.bfloat16)
    cin, HW = x.shape
    cout = wkw.shape[1]
    if k == 1:
        acc = jnp.dot(wkw[0], x, preferred_element_type=jnp.float32)
    else:
        pad = (k - 1) // 2 * dil
        ext = pad + 1
        z = jnp.zeros((cin, ext * W), x.dtype)
        xp = jnp.concatenate([z, x, z], axis=1)
        col = jax.lax.broadcasted_iota(jnp.int32, (1, HW), 1) % W
        acc = jnp.zeros((cout, HW), jnp.float32)
        for kw in range(k):
            dw = kw * dil - pad
            rows = [xp[:, (ext + kh * dil - pad) * W + dw:
                       (ext + kh * dil - pad) * W + dw + HW]
                    for kh in range(k)]
            patch = jnp.concatenate(rows, axis=0)
            part = jnp.dot(wkw[kw], patch, preferred_element_type=jnp.float32)
            if dw != 0:
                msk = ((col + dw) >= 0) & ((col + dw) < W)
                part = part * msk.astype(part.dtype)
            acc = acc + part
    acc = acc + b
    if relu:
        acc = jnp.maximum(acc, 0.0)
    return acc


def _mega_kernel(y_ref, imap_ref, x_ref, mt_ref, wup_ref, bup_ref,
                 crw_ref, crb_ref,
                 p1w, p1b, d1w, d1b, p2w, p2b, d2w, d2b,
                 p3w, p3b, d3w, d3b, p4w, p4b, d4w, d4b,
                 fw, fb, rp_ref, wz_ref, ob_ref,
                 r2_ref, om_ref, *, w_small, W, cs):
    c1, HW = r2_ref.shape

    up_small = _conv_hw(y_ref[...], wup_ref[...], bup_ref[...],
                        k=7, dil=1, W=w_small)
    src = jnp.concatenate([up_small, imap_ref[...]], axis=0)
    big = jnp.dot(src.astype(jnp.bfloat16), mt_ref[...],
                  preferred_element_type=jnp.float32)
    up = big[:c1]
    mval = jax.nn.sigmoid(big[c1:c1 + 1])

    x = x_ref[...]
    outs = []
    for br in range(2):
        feat = x * (mval if br == 0 else (1.0 - mval))
        cr = jnp.maximum(
            jnp.dot(crw_ref[br], feat.astype(jnp.bfloat16),
                    preferred_element_type=jnp.float32) + crb_ref[br], 0.0)
        p1 = _conv_hw(cr[0 * cs:1 * cs], p1w[br], p1b[br], k=1, dil=1, W=W)
        d1 = _conv_hw(p1, d1w[br], d1b[br], k=3, dil=1, W=W)
        p2 = _conv_hw(cr[1 * cs:2 * cs] + d1, p2w[br], p2b[br],
                      k=3, dil=1, W=W)
        d2 = _conv_hw(p2, d2w[br], d2b[br], k=3, dil=2, W=W)
        p3 = _conv_hw(cr[2 * cs:3 * cs] + d2, p3w[br], p3b[br],
                      k=5, dil=1, W=W)
        d3 = _conv_hw(p3, d3w[br], d3b[br], k=3, dil=4, W=W)
        p4 = _conv_hw(cr[3 * cs:4 * cs] + d3, p4w[br], p4b[br],
                      k=7, dil=1, W=W)
        d4 = _conv_hw(p4, d4w[br], d4b[br], k=3, dil=8, W=W)
        cat = jnp.concatenate([d1, d2, d3, d4], axis=0)
        outs.append(_conv_hw(cat, fw[br], fb[br], k=1, dil=1, W=W))
    fp_o, fn_o = outs

    rp = rp_ref[...]
    r1 = jnp.maximum(rp[0] * up + rp[1] * fp_o + rp[2], 0.0)
    r2 = jnp.maximum(rp[3] * r1 + rp[4] * fn_o + rp[5], 0.0)
    r2_ref[...] = r2
    ext = 4
    r2h = r2.astype(jnp.bfloat16)
    z = jnp.zeros((c1, ext * W), r2h.dtype)
    r2p = jnp.concatenate([z, r2h, z], axis=1)
    Z = jnp.dot(wz_ref[...], r2p, preferred_element_type=jnp.float32)
    col = jax.lax.broadcasted_iota(jnp.int32, (1, HW), 1) % W
    acc = jnp.zeros((1, HW), jnp.float32) + ob_ref[0, 0]
    for kh in range(7):
        for kw in range(7):
            dw = kw - 3
            s = (ext + kh - 3) * W + dw
            part = Z[kh * 7 + kw:kh * 7 + kw + 1, s:s + HW]
            if dw != 0:
                msk = ((col + dw) >= 0) & ((col + dw) < W)
                part = part * msk.astype(part.dtype)
            acc = acc + part
    om_ref[...] = acc


def kernel(x, y, in_map, up__w, up__b, up__gamma, up__beta, up__mean, up__var, up2__w, up2__b, up2__gamma, up2__beta, up2__mean, up2__var, output_map__w, output_map__b, fp__cr1__w, fp__cr1__b, fp__cr1__gamma, fp__cr1__beta, fp__cr1__mean, fp__cr1__var, fp__cr2__w, fp__cr2__b, fp__cr2__gamma, fp__cr2__beta, fp__cr2__mean, fp__cr2__var, fp__cr3__w, fp__cr3__b, fp__cr3__gamma, fp__cr3__beta, fp__cr3__mean, fp__cr3__var, fp__cr4__w, fp__cr4__b, fp__cr4__gamma, fp__cr4__beta, fp__cr4__mean, fp__cr4__var, fp__p1__w, fp__p1__b, fp__p1__gamma, fp__p1__beta, fp__p1__mean, fp__p1__var, fp__p1_dc__w, fp__p1_dc__b, fp__p1_dc__gamma, fp__p1_dc__beta, fp__p1_dc__mean, fp__p1_dc__var, fp__p2__w, fp__p2__b, fp__p2__gamma, fp__p2__beta, fp__p2__mean, fp__p2__var, fp__p2_dc__w, fp__p2_dc__b, fp__p2_dc__gamma, fp__p2_dc__beta, fp__p2_dc__mean, fp__p2_dc__var, fp__p3__w, fp__p3__b, fp__p3__gamma, fp__p3__beta, fp__p3__mean, fp__p3__var, fp__p3_dc__w, fp__p3_dc__b, fp__p3_dc__gamma, fp__p3_dc__beta, fp__p3_dc__mean, fp__p3_dc__var, fp__p4__w, fp__p4__b, fp__p4__gamma, fp__p4__beta, fp__p4__mean, fp__p4__var, fp__p4_dc__w, fp__p4_dc__b, fp__p4_dc__gamma, fp__p4_dc__beta, fp__p4_dc__mean, fp__p4_dc__var, fp__fusion__w, fp__fusion__b, fp__fusion__gamma, fp__fusion__beta, fp__fusion__mean, fp__fusion__var, fn__cr1__w, fn__cr1__b, fn__cr1__gamma, fn__cr1__beta, fn__cr1__mean, fn__cr1__var, fn__cr2__w, fn__cr2__b, fn__cr2__gamma, fn__cr2__beta, fn__cr2__mean, fn__cr2__var, fn__cr3__w, fn__cr3__b, fn__cr3__gamma, fn__cr3__beta, fn__cr3__mean, fn__cr3__var, fn__cr4__w, fn__cr4__b, fn__cr4__gamma, fn__cr4__beta, fn__cr4__mean, fn__cr4__var, fn__p1__w, fn__p1__b, fn__p1__gamma, fn__p1__beta, fn__p1__mean, fn__p1__var, fn__p1_dc__w, fn__p1_dc__b, fn__p1_dc__gamma, fn__p1_dc__beta, fn__p1_dc__mean, fn__p1_dc__var, fn__p2__w, fn__p2__b, fn__p2__gamma, fn__p2__beta, fn__p2__mean, fn__p2__var, fn__p2_dc__w, fn__p2_dc__b, fn__p2_dc__gamma, fn__p2_dc__beta, fn__p2_dc__mean, fn__p2_dc__var, fn__p3__w, fn__p3__b, fn__p3__gamma, fn__p3__beta, fn__p3__mean, fn__p3__var, fn__p3_dc__w, fn__p3_dc__b, fn__p3_dc__gamma, fn__p3_dc__beta, fn__p3_dc__mean, fn__p3_dc__var, fn__p4__w, fn__p4__b, fn__p4__gamma, fn__p4__beta, fn__p4__mean, fn__p4__var, fn__p4_dc__w, fn__p4_dc__b, fn__p4_dc__gamma, fn__p4_dc__beta, fn__p4_dc__mean, fn__p4_dc__var, fn__fusion__w, fn__fusion__b, fn__fusion__gamma, fn__fusion__beta, fn__fusion__mean, fn__fusion__var, bn1__gamma, bn1__beta, bn1__mean, bn1__var, bn2__gamma, bn2__beta, bn2__mean, bn2__var, alpha, beta):
    N, C1, H, W = x.shape
    C2 = y.shape[1]
    h, w = H // 2, W // 2
    HW, hw = H * W, h * w
    cs = C1 // 4

    wup, bup = _prep_conv(*_fold_bn(up__w, up__b, up__gamma, up__beta,
                                    up__mean, up__var))
    MT = jnp.asarray(np.kron(_bilin_mat(h, H), _bilin_mat(w, W)).T
                     ).astype(jnp.bfloat16)

    branches = (
        dict(cr1=(fp__cr1__w, fp__cr1__b, fp__cr1__gamma, fp__cr1__beta, fp__cr1__mean, fp__cr1__var),
             cr2=(fp__cr2__w, fp__cr2__b, fp__cr2__gamma, fp__cr2__beta, fp__cr2__mean, fp__cr2__var),
             cr3=(fp__cr3__w, fp__cr3__b, fp__cr3__gamma, fp__cr3__beta, fp__cr3__mean, fp__cr3__var),
             cr4=(fp__cr4__w, fp__cr4__b, fp__cr4__gamma, fp__cr4__beta, fp__cr4__mean, fp__cr4__var),
             p1=(fp__p1__w, fp__p1__b, fp__p1__gamma, fp__p1__beta, fp__p1__mean, fp__p1__var),
             p1_dc=(fp__p1_dc__w, fp__p1_dc__b, fp__p1_dc__gamma, fp__p1_dc__beta, fp__p1_dc__mean, fp__p1_dc__var),
             p2=(fp__p2__w, fp__p2__b, fp__p2__gamma, fp__p2__beta, fp__p2__mean, fp__p2__var),
             p2_dc=(fp__p2_dc__w, fp__p2_dc__b, fp__p2_dc__gamma, fp__p2_dc__beta, fp__p2_dc__mean, fp__p2_dc__var),
             p3=(fp__p3__w, fp__p3__b, fp__p3__gamma, fp__p3__beta, fp__p3__mean, fp__p3__var),
             p3_dc=(fp__p3_dc__w, fp__p3_dc__b, fp__p3_dc__gamma, fp__p3_dc__beta, fp__p3_dc__mean, fp__p3_dc__var),
             p4=(fp__p4__w, fp__p4__b, fp__p4__gamma, fp__p4__beta, fp__p4__mean, fp__p4__var),
             p4_dc=(fp__p4_dc__w, fp__p4_dc__b, fp__p4_dc__gamma, fp__p4_dc__beta, fp__p4_dc__mean, fp__p4_dc__var),
             fusion=(fp__fusion__w, fp__fusion__b, fp__fusion__gamma, fp__fusion__beta, fp__fusion__mean, fp__fusion__var)),
        dict(cr1=(fn__cr1__w, fn__cr1__b, fn__cr1__gamma, fn__cr1__beta, fn__cr1__mean, fn__cr1__var),
             cr2=(fn__cr2__w, fn__cr2__b, fn__cr2__gamma, fn__cr2__beta, fn__cr2__mean, fn__cr2__var),
             cr3=(fn__cr3__w, fn__cr3__b, fn__cr3__gamma, fn__cr3__beta, fn__cr3__mean, fn__cr3__var),
             cr4=(fn__cr4__w, fn__cr4__b, fn__cr4__gamma, fn__cr4__beta, fn__cr4__mean, fn__cr4__var),
             p1=(fn__p1__w, fn__p1__b, fn__p1__gamma, fn__p1__beta, fn__p1__mean, fn__p1__var),
             p1_dc=(fn__p1_dc__w, fn__p1_dc__b, fn__p1_dc__gamma, fn__p1_dc__beta, fn__p1_dc__mean, fn__p1_dc__var),
             p2=(fn__p2__w, fn__p2__b, fn__p2__gamma, fn__p2__beta, fn__p2__mean, fn__p2__var),
             p2_dc=(fn__p2_dc__w, fn__p2_dc__b, fn__p2_dc__gamma, fn__p2_dc__beta, fn__p2_dc__mean, fn__p2_dc__var),
             p3=(fn__p3__w, fn__p3__b, fn__p3__gamma, fn__p3__beta, fn__p3__mean, fn__p3__var),
             p3_dc=(fn__p3_dc__w, fn__p3_dc__b, fn__p3_dc__gamma, fn__p3_dc__beta, fn__p3_dc__mean, fn__p3_dc__var),
             p4=(fn__p4__w, fn__p4__b, fn__p4__gamma, fn__p4__beta, fn__p4__mean, fn__p4__var),
             p4_dc=(fn__p4_dc__w, fn__p4_dc__b, fn__p4_dc__gamma, fn__p4_dc__beta, fn__p4_dc__mean, fn__p4_dc__var),
             fusion=(fn__fusion__w, fn__fusion__b, fn__fusion__gamma, fn__fusion__beta, fn__fusion__mean, fn__fusion__var)))

    def folded(br, nm):
        return _fold_bn(*br[nm])

    crw, crb = [], []
    for br in branches:
        ws, bs = [], []
        for nm in ("cr1", "cr2", "cr3", "cr4"):
            wf, bf = folded(br, nm)
            ws.append(wf.reshape(cs, C1))
            bs.append(bf)
        crw.append(jnp.concatenate(ws, axis=0))
        crb.append(jnp.concatenate(bs, axis=0).reshape(C1, 1))
    crw = jnp.stack(crw).astype(jnp.bfloat16)
    crb = jnp.stack(crb)

    def stacked(nm):
        pw, pb = _prep_conv(*folded(branches[0], nm))
        nw, nb = _prep_conv(*folded(branches[1], nm))
        return jnp.stack([pw, nw]), jnp.stack([pb, nb])

    conv_names = ("p1", "p1_dc", "p2", "p2_dc", "p3", "p3_dc",
                  "p4", "p4_dc", "fusion")
    packed = [a for nm in conv_names for a in stacked(nm)]

    s1 = bn1__gamma * jax.lax.rsqrt(bn1__var + _BN_EPS)
    b1 = bn1__beta - bn1__mean * s1
    s2 = bn2__gamma * jax.lax.rsqrt(bn2__var + _BN_EPS)
    b2 = bn2__beta - bn2__mean * s2
    rparams = jnp.stack([s1, -alpha[0] * s1, b1,
                         s2, beta[0] * s2, b2]).reshape(6, C1, 1)
    wz = jnp.transpose(output_map__w[0], (1, 2, 0)).reshape(49, C1)
    wz = wz.astype(jnp.bfloat16)
    ob = output_map__b.reshape(1, 1)

    y2 = y.reshape(N, C2, hw)
    imap2 = in_map.reshape(N, 1, hw)
    x2 = x.reshape(N, C1, HW)

    consts = [MT, wup, bup, crw, crb] + packed + [rparams, wz, ob]
    cspecs = [pl.BlockSpec(a.shape, lambda n, nd=a.ndim: (0,) * nd)
              for a in consts]

    r2_flat, om_flat = pl.pallas_call(
        functools.partial(_mega_kernel, w_small=w, W=W, cs=cs),
        out_shape=(jax.ShapeDtypeStruct((N, C1, HW), jnp.float32),
                   jax.ShapeDtypeStruct((N, 1, HW), jnp.float32)),
        grid=(N,),
        in_specs=[pl.BlockSpec((None, C2, hw), lambda n: (n, 0, 0)),
                  pl.BlockSpec((None, 1, hw), lambda n: (n, 0, 0)),
                  pl.BlockSpec((None, C1, HW), lambda n: (n, 0, 0))] + cspecs,
        out_specs=(pl.BlockSpec((None, C1, HW), lambda n: (n, 0, 0)),
                   pl.BlockSpec((None, 1, HW), lambda n: (n, 0, 0))),
        compiler_params=_PAR,
    )(y2, imap2, x2, *consts)

    return r2_flat.reshape(N, C1, H, W), om_flat.reshape(N, 1, H, W)
